# baseline (speedup 1.0000x reference)
import functools

import jax
import jax.numpy as jnp
from jax import lax
from jax.experimental import pallas as pl
from jax.experimental.pallas import tpu as pltpu
from jax.experimental.pallas import tpu_sc as plsc

N_ROWS = 1048576
D = 64
NC = 2
NS = 16
NW = NC * NS
BLK = 8192
N_SC = 524288
TC_OFF = N_SC // BLK
G_TC = (N_ROWS - N_SC) // BLK
ROWS_W = N_SC // NW
CHUNK = 128
NCH = ROWS_W // CHUNK
GROUPS = CHUNK // 16


def _sqrt16(x):
    x = jnp.maximum(x, 0.0)
    i = plsc.bitcast(x, jnp.int32)
    y = plsc.bitcast(jnp.int32(0x5F3759DF) - (i >> 1), jnp.float32)
    for _ in range(3):
        y = y * (1.5 - 0.5 * x * y * y)
    return x * y


def _sc_partials(f, t, center):
    mesh = plsc.VectorSubcoreMesh(core_axis_name="c", subcore_axis_name="s")

    @functools.partial(
        pl.kernel,
        mesh=mesh,
        compiler_params=pltpu.CompilerParams(needs_layout_passes=False),
        out_type=jax.ShapeDtypeStruct((NW, 48), jnp.float32),
        scratch_types=[
            pltpu.VMEM((CHUNK, D), jnp.float32),
            pltpu.VMEM((CHUNK, D), jnp.float32),
            pltpu.VMEM((CHUNK,), jnp.int32),
            pltpu.VMEM((CHUNK,), jnp.int32),
            pltpu.VMEM((2, D), jnp.float32),
            pltpu.VMEM((2 * D,), jnp.float32),
            pltpu.VMEM((2 * D,), jnp.float32),
            pltpu.VMEM((48,), jnp.float32),
            pltpu.SemaphoreType.DMA,
            pltpu.SemaphoreType.DMA,
            pltpu.SemaphoreType.DMA,
        ],
    )
    def k(f_hbm, t_hbm, c_hbm, out_hbm, fb0, fb1, tb0, tb1, cv, c0b, dlb,
          ov, sem0, sem1, semc):
        wid = lax.axis_index("s") * NC + lax.axis_index("c")
        base = wid * ROWS_W

        pltpu.async_copy(c_hbm, cv, semc).wait()

        def start(g, fbuf, tbuf, sem):
            row0 = base + g * CHUNK
            cp_f = pltpu.async_copy(
                f_hbm.at[pl.ds(row0, CHUNK)], fbuf, sem)
            cp_t = pltpu.async_copy(
                t_hbm.at[pl.ds(row0, CHUNK)], tbuf, sem)
            return cp_f, cp_t

        def wait(fbuf, tbuf, sem):
            pltpu.make_async_copy(
                f_hbm.at[pl.ds(0, CHUNK)], fbuf, sem).wait()
            pltpu.make_async_copy(
                t_hbm.at[pl.ds(0, CHUNK)], tbuf, sem).wait()

        lanes = lax.iota(jnp.int32, 16)
        c0vecs = [cv[0, pl.ds(p * 16, 16)] for p in range(D // 16)]
        c1vecs = [cv[1, pl.ds(p * 16, 16)] for p in range(D // 16)]
        c0s = [c0vecs[kk // 16][kk % 16] for kk in range(D)]
        dels = [c1vecs[kk // 16][kk % 16] - c0s[kk] for kk in range(D)]
        cc = jnp.zeros((), jnp.float32)
        for kk in range(D):
            cc = cc + dels[kk] * dels[kk]
        for p in range(D // 16):
            c0p = c0vecs[p]
            dvp = c1vecs[p] - c0vecs[p]
            c0b[pl.ds(p * 16, 16)] = c0p
            c0b[pl.ds(D + p * 16, 16)] = c0p
            dlb[pl.ds(p * 16, 16)] = dvp
            dlb[pl.ds(D + p * 16, 16)] = dvp

        def compute(fbuf, tbuf, carry):
            s0, s1, n1 = carry
            z16 = jnp.zeros((16,), jnp.float32)
            rows = [j * 16 + lanes for j in range(GROUPS)]

            def kblock(kb, ab):
                a, b = map(list, ab)
                for kk2 in range(8):
                    kk = kb * 8 + kk2
                    col = (lanes + kk) & (D - 1)
                    c0v = plsc.load_gather(c0b, [col])
                    dv = plsc.load_gather(dlb, [col])
                    for j in range(GROUPS):
                        fv = plsc.load_gather(fbuf, [rows[j], col])
                        g = fv - c0v
                        a[j] = a[j] + g * g
                        b[j] = b[j] + g * dv
                return tuple(a), tuple(b)

            a, b = lax.fori_loop(
                0, D // 8, kblock,
                (tuple([z16] * GROUPS), tuple([z16] * GROUPS)))
            zero = jnp.zeros((16,), jnp.float32)
            one = jnp.ones((16,), jnp.float32)
            for j in range(GROUPS):
                tvec = tbuf[pl.ds(j * 16, 16)]
                m = tvec == 1
                d0 = _sqrt16(a[j])
                d1 = _sqrt16(a[j] - 2.0 * b[j] + cc)
                s0 = s0 + jnp.where(m, zero, d0)
                s1 = s1 + jnp.where(m, d1, zero)
                n1 = n1 + jnp.where(m, one, zero)
            return s0, s1, n1

        z16 = jnp.zeros((16,), jnp.float32)
        carry = (z16, z16, z16)

        start(0, fb0, tb0, sem0)
        start(1, fb1, tb1, sem1)

        def pair(g2, carry):
            wait(fb0, tb0, sem0)
            carry = compute(fb0, tb0, carry)
            start(2 * g2 + 2, fb0, tb0, sem0)
            wait(fb1, tb1, sem1)
            carry = compute(fb1, tb1, carry)
            start(2 * g2 + 3, fb1, tb1, sem1)
            return carry

        carry = lax.fori_loop(0, NCH // 2 - 1, pair, carry)
        wait(fb0, tb0, sem0)
        carry = compute(fb0, tb0, carry)
        wait(fb1, tb1, sem1)
        carry = compute(fb1, tb1, carry)

        s0a, s1a, n1a = carry
        ov[pl.ds(0, 16)] = s0a
        ov[pl.ds(16, 16)] = s1a
        ov[pl.ds(32, 16)] = n1a
        pltpu.sync_copy(ov, out_hbm.at[wid])

    return k(f, t, center)


def _tc_body(t_ref, f_ref, c_ref, out_ref, acc_ref):
    i = pl.program_id(0)
    g = pl.num_programs(0)

    @pl.when(i == 0)
    def _init():
        acc_ref[0] = 0.0
        acc_ref[1] = 0.0
        acc_ref[2] = 0.0

    tf = t_ref[pl.ds(TC_OFF + i, 1), :].T
    fb = f_ref[...]
    c0 = c_ref[0:1, :]
    c1 = c_ref[1:2, :]
    csel = jnp.where(tf == 1.0, c1, c0)
    diff = fb - csel
    d = jnp.sqrt(jnp.sum(diff * diff, axis=1, keepdims=True))
    s1 = jnp.sum(d * tf)
    acc_ref[0] += jnp.sum(d) - s1
    acc_ref[1] += s1
    acc_ref[2] += jnp.sum(tf)

    @pl.when(i == g - 1)
    def _fin():
        out_ref[0, 0] = acc_ref[0]
        out_ref[0, 1] = acc_ref[1]
        out_ref[0, 2] = acc_ref[2]
        out_ref[0, 3] = 0.0


def _tc_partials(f, t2, center):
    return pl.pallas_call(
        _tc_body,
        grid=(G_TC,),
        in_specs=[
            pl.BlockSpec((N_ROWS // BLK, BLK), lambda i: (0, 0)),
            pl.BlockSpec((BLK, D), lambda i: (TC_OFF + i, 0)),
            pl.BlockSpec((2, D), lambda i: (0, 0)),
        ],
        out_specs=pl.BlockSpec(
            (1, 4), lambda i: (0, 0), memory_space=pltpu.SMEM
        ),
        out_shape=jax.ShapeDtypeStruct((1, 4), jnp.float32),
        scratch_shapes=[pltpu.SMEM((4,), jnp.float32)],
    )(t2, f, center)


@jax.jit
def kernel(f, t, center):
    t2 = t.astype(jnp.float32).reshape(N_ROWS // BLK, BLK)
    p = _sc_partials(f, t, center)
    q = _tc_partials(f, t2, center)
    s0 = jnp.sum(p[:, 0:16]) + q[0, 0]
    s1 = jnp.sum(p[:, 16:32]) + q[0, 1]
    n1 = jnp.sum(p[:, 32:48]) + q[0, 2]
    n0 = jnp.float32(N_ROWS) - n1
    r0 = jnp.where(n0 > 0.0, s0 / n0, 0.0)
    r1 = jnp.where(n1 > 0.0, s1 / n1, 0.0)
    return r0 + r1


# speedup vs baseline: 1.0457x; 1.0457x over previous
import functools

import jax
import jax.numpy as jnp
from jax import lax
from jax.experimental import pallas as pl
from jax.experimental.pallas import tpu as pltpu
from jax.experimental.pallas import tpu_sc as plsc

N_ROWS = 1048576
D = 64
NC = 2
NS = 16
NW = NC * NS
BLK = 8192
N_SC = 573440
TC_OFF = N_SC // BLK
G_TC = (N_ROWS - N_SC) // BLK
ROWS_W = N_SC // NW
CHUNK = 128
NCH = ROWS_W // CHUNK
GROUPS = CHUNK // 16


def _sqrt16(x):
    x = jnp.maximum(x, 0.0)
    i = plsc.bitcast(x, jnp.int32)
    y = plsc.bitcast(jnp.int32(0x5F3759DF) - (i >> 1), jnp.float32)
    for _ in range(3):
        y = y * (1.5 - 0.5 * x * y * y)
    return x * y


def _sc_partials(f, t, center):
    mesh = plsc.VectorSubcoreMesh(core_axis_name="c", subcore_axis_name="s")

    @functools.partial(
        pl.kernel,
        mesh=mesh,
        compiler_params=pltpu.CompilerParams(needs_layout_passes=False),
        out_type=jax.ShapeDtypeStruct((NW, 48), jnp.float32),
        scratch_types=[
            pltpu.VMEM((CHUNK, D), jnp.float32),
            pltpu.VMEM((CHUNK, D), jnp.float32),
            pltpu.VMEM((CHUNK,), jnp.int32),
            pltpu.VMEM((CHUNK,), jnp.int32),
            pltpu.VMEM((2, D), jnp.float32),
            pltpu.VMEM((2 * D,), jnp.float32),
            pltpu.VMEM((2 * D,), jnp.float32),
            pltpu.VMEM((48,), jnp.float32),
            pltpu.SemaphoreType.DMA,
            pltpu.SemaphoreType.DMA,
            pltpu.SemaphoreType.DMA,
        ],
    )
    def k(f_hbm, t_hbm, c_hbm, out_hbm, fb0, fb1, tb0, tb1, cv, c0b, dlb,
          ov, sem0, sem1, semc):
        wid = lax.axis_index("s") * NC + lax.axis_index("c")
        base = wid * ROWS_W

        pltpu.async_copy(c_hbm, cv, semc).wait()

        def start(g, fbuf, tbuf, sem):
            row0 = base + g * CHUNK
            cp_f = pltpu.async_copy(
                f_hbm.at[pl.ds(row0, CHUNK)], fbuf, sem)
            cp_t = pltpu.async_copy(
                t_hbm.at[pl.ds(row0, CHUNK)], tbuf, sem)
            return cp_f, cp_t

        def wait(fbuf, tbuf, sem):
            pltpu.make_async_copy(
                f_hbm.at[pl.ds(0, CHUNK)], fbuf, sem).wait()
            pltpu.make_async_copy(
                t_hbm.at[pl.ds(0, CHUNK)], tbuf, sem).wait()

        lanes = lax.iota(jnp.int32, 16)
        c0vecs = [cv[0, pl.ds(p * 16, 16)] for p in range(D // 16)]
        c1vecs = [cv[1, pl.ds(p * 16, 16)] for p in range(D // 16)]
        c0s = [c0vecs[kk // 16][kk % 16] for kk in range(D)]
        dels = [c1vecs[kk // 16][kk % 16] - c0s[kk] for kk in range(D)]
        cc = jnp.zeros((), jnp.float32)
        for kk in range(D):
            cc = cc + dels[kk] * dels[kk]
        for p in range(D // 16):
            c0p = c0vecs[p]
            dvp = c1vecs[p] - c0vecs[p]
            c0b[pl.ds(p * 16, 16)] = c0p
            c0b[pl.ds(D + p * 16, 16)] = c0p
            dlb[pl.ds(p * 16, 16)] = dvp
            dlb[pl.ds(D + p * 16, 16)] = dvp

        def compute(fbuf, tbuf, carry):
            s0, s1, n1 = carry
            z16 = jnp.zeros((16,), jnp.float32)
            rows = [j * 16 + lanes for j in range(GROUPS)]

            def kblock(kb, ab):
                a, b = map(list, ab)
                for kk2 in range(8):
                    kk = kb * 8 + kk2
                    col = (lanes + kk) & (D - 1)
                    c0v = plsc.load_gather(c0b, [col])
                    dv = plsc.load_gather(dlb, [col])
                    for j in range(GROUPS):
                        fv = plsc.load_gather(fbuf, [rows[j], col])
                        g = fv - c0v
                        a[j] = a[j] + g * g
                        b[j] = b[j] + g * dv
                return tuple(a), tuple(b)

            a, b = lax.fori_loop(
                0, D // 8, kblock,
                (tuple([z16] * GROUPS), tuple([z16] * GROUPS)))
            zero = jnp.zeros((16,), jnp.float32)
            one = jnp.ones((16,), jnp.float32)
            for j in range(GROUPS):
                tvec = tbuf[pl.ds(j * 16, 16)]
                m = tvec == 1
                d0 = _sqrt16(a[j])
                d1 = _sqrt16(a[j] - 2.0 * b[j] + cc)
                s0 = s0 + jnp.where(m, zero, d0)
                s1 = s1 + jnp.where(m, d1, zero)
                n1 = n1 + jnp.where(m, one, zero)
            return s0, s1, n1

        z16 = jnp.zeros((16,), jnp.float32)
        carry = (z16, z16, z16)

        start(0, fb0, tb0, sem0)
        start(1, fb1, tb1, sem1)

        def pair(g2, carry):
            wait(fb0, tb0, sem0)
            carry = compute(fb0, tb0, carry)
            start(2 * g2 + 2, fb0, tb0, sem0)
            wait(fb1, tb1, sem1)
            carry = compute(fb1, tb1, carry)
            start(2 * g2 + 3, fb1, tb1, sem1)
            return carry

        carry = lax.fori_loop(0, NCH // 2 - 1, pair, carry)
        wait(fb0, tb0, sem0)
        carry = compute(fb0, tb0, carry)
        wait(fb1, tb1, sem1)
        carry = compute(fb1, tb1, carry)

        s0a, s1a, n1a = carry
        ov[pl.ds(0, 16)] = s0a
        ov[pl.ds(16, 16)] = s1a
        ov[pl.ds(32, 16)] = n1a
        pltpu.sync_copy(ov, out_hbm.at[wid])

    return k(f, t, center)


def _tc_body(t_ref, f_ref, c_ref, out_ref, acc_ref):
    i = pl.program_id(0)
    g = pl.num_programs(0)

    @pl.when(i == 0)
    def _init():
        acc_ref[0] = 0.0
        acc_ref[1] = 0.0
        acc_ref[2] = 0.0

    trow = t_ref[pl.ds(TC_OFF + i, 1), :].astype(jnp.float32)
    tf = trow.T
    fb = f_ref[...]
    c0 = c_ref[0:1, :]
    c1 = c_ref[1:2, :]
    csel = jnp.where(tf == 1.0, c1, c0)
    diff = fb - csel
    d = jnp.sqrt(jnp.sum(diff * diff, axis=1, keepdims=True))
    s1 = jnp.sum(d * tf)
    acc_ref[0] += jnp.sum(d) - s1
    acc_ref[1] += s1
    acc_ref[2] += jnp.sum(tf)

    @pl.when(i == g - 1)
    def _fin():
        out_ref[0, 0] = acc_ref[0]
        out_ref[0, 1] = acc_ref[1]
        out_ref[0, 2] = acc_ref[2]
        out_ref[0, 3] = 0.0


def _tc_partials(f, t2, center):
    return pl.pallas_call(
        _tc_body,
        grid=(G_TC,),
        in_specs=[
            pl.BlockSpec((N_ROWS // BLK, BLK), lambda i: (0, 0)),
            pl.BlockSpec((BLK, D), lambda i: (TC_OFF + i, 0)),
            pl.BlockSpec((2, D), lambda i: (0, 0)),
        ],
        out_specs=pl.BlockSpec(
            (1, 4), lambda i: (0, 0), memory_space=pltpu.SMEM
        ),
        out_shape=jax.ShapeDtypeStruct((1, 4), jnp.float32),
        scratch_shapes=[pltpu.SMEM((4,), jnp.float32)],
    )(t2, f, center)


@jax.jit
def kernel(f, t, center):
    t2 = t.reshape(N_ROWS // BLK, BLK)
    p = _sc_partials(f, t, center)
    q = _tc_partials(f, t2, center)
    s0 = jnp.sum(p[:, 0:16]) + q[0, 0]
    s1 = jnp.sum(p[:, 16:32]) + q[0, 1]
    n1 = jnp.sum(p[:, 32:48]) + q[0, 2]
    n0 = jnp.float32(N_ROWS) - n1
    r0 = jnp.where(n0 > 0.0, s0 / n0, 0.0)
    r1 = jnp.where(n1 > 0.0, s1 / n1, 0.0)
    return r0 + r1


# speedup vs baseline: 1.0601x; 1.0138x over previous
import functools

import jax
import jax.numpy as jnp
from jax import lax
from jax.experimental import pallas as pl
from jax.experimental.pallas import tpu as pltpu
from jax.experimental.pallas import tpu_sc as plsc

N_ROWS = 1048576
D = 64
NC = 2
NS = 16
NW = NC * NS
BLK = 8192
N_SC = 622592
TC_OFF = N_SC // BLK
G_TC = (N_ROWS - N_SC) // BLK
ROWS_W = N_SC // NW
CHUNK = 128
NCH = ROWS_W // CHUNK
GROUPS = CHUNK // 16


def _sqrt16(x):
    x = jnp.maximum(x, 0.0)
    i = plsc.bitcast(x, jnp.int32)
    y = plsc.bitcast(jnp.int32(0x5F3759DF) - (i >> 1), jnp.float32)
    for _ in range(3):
        y = y * (1.5 - 0.5 * x * y * y)
    return x * y


def _sc_partials(f, t, center):
    mesh = plsc.VectorSubcoreMesh(core_axis_name="c", subcore_axis_name="s")

    @functools.partial(
        pl.kernel,
        mesh=mesh,
        compiler_params=pltpu.CompilerParams(needs_layout_passes=False),
        out_type=jax.ShapeDtypeStruct((NW, 48), jnp.float32),
        scratch_types=[
            pltpu.VMEM((CHUNK, D), jnp.float32),
            pltpu.VMEM((CHUNK, D), jnp.float32),
            pltpu.VMEM((CHUNK,), jnp.int32),
            pltpu.VMEM((CHUNK,), jnp.int32),
            pltpu.VMEM((2, D), jnp.float32),
            pltpu.VMEM((2 * D,), jnp.float32),
            pltpu.VMEM((2 * D,), jnp.float32),
            pltpu.VMEM((48,), jnp.float32),
            pltpu.SemaphoreType.DMA,
            pltpu.SemaphoreType.DMA,
            pltpu.SemaphoreType.DMA,
        ],
    )
    def k(f_hbm, t_hbm, c_hbm, out_hbm, fb0, fb1, tb0, tb1, cv, c0b, dlb,
          ov, sem0, sem1, semc):
        wid = lax.axis_index("s") * NC + lax.axis_index("c")
        base = wid * ROWS_W

        pltpu.async_copy(c_hbm, cv, semc).wait()

        def start(g, fbuf, tbuf, sem):
            row0 = base + g * CHUNK
            cp_f = pltpu.async_copy(
                f_hbm.at[pl.ds(row0, CHUNK)], fbuf, sem)
            cp_t = pltpu.async_copy(
                t_hbm.at[pl.ds(row0, CHUNK)], tbuf, sem)
            return cp_f, cp_t

        def wait(fbuf, tbuf, sem):
            pltpu.make_async_copy(
                f_hbm.at[pl.ds(0, CHUNK)], fbuf, sem).wait()
            pltpu.make_async_copy(
                t_hbm.at[pl.ds(0, CHUNK)], tbuf, sem).wait()

        lanes = lax.iota(jnp.int32, 16)
        c0vecs = [cv[0, pl.ds(p * 16, 16)] for p in range(D // 16)]
        c1vecs = [cv[1, pl.ds(p * 16, 16)] for p in range(D // 16)]
        c0s = [c0vecs[kk // 16][kk % 16] for kk in range(D)]
        dels = [c1vecs[kk // 16][kk % 16] - c0s[kk] for kk in range(D)]
        cc = jnp.zeros((), jnp.float32)
        for kk in range(D):
            cc = cc + dels[kk] * dels[kk]
        for p in range(D // 16):
            c0p = c0vecs[p]
            dvp = c1vecs[p] - c0vecs[p]
            c0b[pl.ds(p * 16, 16)] = c0p
            c0b[pl.ds(D + p * 16, 16)] = c0p
            dlb[pl.ds(p * 16, 16)] = dvp
            dlb[pl.ds(D + p * 16, 16)] = dvp

        def compute(fbuf, tbuf, carry):
            s0, s1, n1 = carry
            z16 = jnp.zeros((16,), jnp.float32)
            rows = [j * 16 + lanes for j in range(GROUPS)]

            def kblock(kb, ab):
                a, b = map(list, ab)
                for kk2 in range(8):
                    kk = kb * 8 + kk2
                    col = (lanes + kk) & (D - 1)
                    c0v = plsc.load_gather(c0b, [col])
                    dv = plsc.load_gather(dlb, [col])
                    for j in range(GROUPS):
                        fv = plsc.load_gather(fbuf, [rows[j], col])
                        g = fv - c0v
                        a[j] = a[j] + g * g
                        b[j] = b[j] + g * dv
                return tuple(a), tuple(b)

            a, b = lax.fori_loop(
                0, D // 8, kblock,
                (tuple([z16] * GROUPS), tuple([z16] * GROUPS)))
            zero = jnp.zeros((16,), jnp.float32)
            one = jnp.ones((16,), jnp.float32)
            for j in range(GROUPS):
                tvec = tbuf[pl.ds(j * 16, 16)]
                m = tvec == 1
                d0 = _sqrt16(a[j])
                d1 = _sqrt16(a[j] - 2.0 * b[j] + cc)
                s0 = s0 + jnp.where(m, zero, d0)
                s1 = s1 + jnp.where(m, d1, zero)
                n1 = n1 + jnp.where(m, one, zero)
            return s0, s1, n1

        z16 = jnp.zeros((16,), jnp.float32)
        carry = (z16, z16, z16)

        start(0, fb0, tb0, sem0)
        start(1, fb1, tb1, sem1)

        def pair(g2, carry):
            wait(fb0, tb0, sem0)
            carry = compute(fb0, tb0, carry)
            start(2 * g2 + 2, fb0, tb0, sem0)
            wait(fb1, tb1, sem1)
            carry = compute(fb1, tb1, carry)
            start(2 * g2 + 3, fb1, tb1, sem1)
            return carry

        carry = lax.fori_loop(0, NCH // 2 - 1, pair, carry)
        wait(fb0, tb0, sem0)
        carry = compute(fb0, tb0, carry)
        wait(fb1, tb1, sem1)
        carry = compute(fb1, tb1, carry)

        s0a, s1a, n1a = carry
        ov[pl.ds(0, 16)] = s0a
        ov[pl.ds(16, 16)] = s1a
        ov[pl.ds(32, 16)] = n1a
        pltpu.sync_copy(ov, out_hbm.at[wid])

    return k(f, t, center)


def _tc_body(t_ref, f_ref, c_ref, out_ref, acc_ref):
    i = pl.program_id(0)
    g = pl.num_programs(0)

    @pl.when(i == 0)
    def _init():
        acc_ref[0] = 0.0
        acc_ref[1] = 0.0
        acc_ref[2] = 0.0

    trow = t_ref[pl.ds(TC_OFF + i, 1), :].astype(jnp.float32)
    tf = trow.T
    fb = f_ref[...]
    c0 = c_ref[0:1, :]
    c1 = c_ref[1:2, :]
    csel = jnp.where(tf == 1.0, c1, c0)
    diff = fb - csel
    d = jnp.sqrt(jnp.sum(diff * diff, axis=1, keepdims=True))
    s1 = jnp.sum(d * tf)
    acc_ref[0] += jnp.sum(d) - s1
    acc_ref[1] += s1
    acc_ref[2] += jnp.sum(tf)

    @pl.when(i == g - 1)
    def _fin():
        out_ref[0, 0] = acc_ref[0]
        out_ref[0, 1] = acc_ref[1]
        out_ref[0, 2] = acc_ref[2]
        out_ref[0, 3] = 0.0


def _tc_partials(f, t2, center):
    return pl.pallas_call(
        _tc_body,
        grid=(G_TC,),
        in_specs=[
            pl.BlockSpec((N_ROWS // BLK, BLK), lambda i: (0, 0)),
            pl.BlockSpec((BLK, D), lambda i: (TC_OFF + i, 0)),
            pl.BlockSpec((2, D), lambda i: (0, 0)),
        ],
        out_specs=pl.BlockSpec(
            (1, 4), lambda i: (0, 0), memory_space=pltpu.SMEM
        ),
        out_shape=jax.ShapeDtypeStruct((1, 4), jnp.float32),
        scratch_shapes=[pltpu.SMEM((4,), jnp.float32)],
    )(t2, f, center)


@jax.jit
def kernel(f, t, center):
    t2 = t.reshape(N_ROWS // BLK, BLK)
    p = _sc_partials(f, t, center)
    q = _tc_partials(f, t2, center)
    s0 = jnp.sum(p[:, 0:16]) + q[0, 0]
    s1 = jnp.sum(p[:, 16:32]) + q[0, 1]
    n1 = jnp.sum(p[:, 32:48]) + q[0, 2]
    n0 = jnp.float32(N_ROWS) - n1
    r0 = jnp.where(n0 > 0.0, s0 / n0, 0.0)
    r1 = jnp.where(n1 > 0.0, s1 / n1, 0.0)
    return r0 + r1
